# baseline (speedup 1.0000x reference)
import jax
import jax.numpy as jnp
from jax import lax
from jax.experimental import pallas as pl
from jax.experimental.pallas import tpu as pltpu

_BM = 8192
_NFULL = 12
_TSTART = _NFULL * _BM
_TROWS = 100000 - _TSTART
_TALIGN = 1664
_TREM = _TROWS - _TALIGN
_DEPTH = 6


def _in_copy(xt_hbm, xbuf, insem, chunk, slot):
    return pltpu.make_async_copy(
        xt_hbm.at[:, pl.ds(chunk * _BM, _BM)], xbuf.at[slot], insem.at[slot])


def _out_copy(o_hbm, obuf, outsem, chunk, slot):
    return pltpu.make_async_copy(
        obuf.at[slot], o_hbm.at[pl.ds(chunk * _BM, _BM), :], outsem.at[slot])


def _dot_t(x, w):
    return lax.dot_general(x, w, dimension_numbers=(((0,), (0,)), ((), ())),
                           preferred_element_type=jnp.float32)


def _pipeline(xt_hbm, xrem_vmem, w_vmem, o_hbm, xbuf, obuf, xtail, otail,
              insem, outsem, tsem):
    w = w_vmem[...]

    tail_in = pltpu.make_async_copy(
        xt_hbm.at[:, pl.ds(_TSTART, _TALIGN)], xtail, tsem.at[1])
    tail_in.start()

    for d in range(_DEPTH):
        _in_copy(xt_hbm, xbuf, insem, d, d).start()

    tail_out = pltpu.make_async_copy(
        otail, o_hbm.at[pl.ds(_TSTART, _TROWS), :], tsem.at[0])
    tail_in.wait()
    otail[pl.ds(0, _TALIGN), :] = _dot_t(xtail[...], w)
    otail[pl.ds(_TALIGN, _TREM), :] = _dot_t(xrem_vmem[...], w)
    tail_out.start()

    def step(i, carry):
        slot = jax.lax.rem(i, _DEPTH)
        _in_copy(xt_hbm, xbuf, insem, i, slot).wait()

        @pl.when(i >= _DEPTH)
        def _():
            _out_copy(o_hbm, obuf, outsem, i - _DEPTH, slot).wait()

        obuf[slot] = _dot_t(xbuf[slot], w)
        _out_copy(o_hbm, obuf, outsem, i, slot).start()

        @pl.when(i + _DEPTH < _NFULL)
        def _():
            _in_copy(xt_hbm, xbuf, insem, i + _DEPTH, slot).start()

        return carry

    jax.lax.fori_loop(0, _NFULL, step, 0)

    for d in range(_DEPTH):
        chunk = _NFULL - _DEPTH + d
        _out_copy(o_hbm, obuf, outsem, chunk, chunk % _DEPTH).wait()
    tail_out.wait()


@jax.jit
def kernel(one_hot_atomic, W):
    m, k = one_hot_atomic.shape
    n = W.shape[1]
    xt = one_hot_atomic.T
    return pl.pallas_call(
        _pipeline,
        in_specs=[
            pl.BlockSpec(memory_space=pltpu.MemorySpace.HBM),
            pl.BlockSpec((k, _TREM), lambda: (0, 0)),
            pl.BlockSpec((k, n), lambda: (0, 0)),
        ],
        out_specs=pl.BlockSpec(memory_space=pltpu.MemorySpace.HBM),
        out_shape=jax.ShapeDtypeStruct((m, n), jnp.float32),
        scratch_shapes=[
            pltpu.VMEM((_DEPTH, k, _BM), jnp.float32),
            pltpu.VMEM((_DEPTH, _BM, n), jnp.float32),
            pltpu.VMEM((k, _TALIGN), jnp.float32),
            pltpu.VMEM((_TROWS, n), jnp.float32),
            pltpu.SemaphoreType.DMA((_DEPTH,)),
            pltpu.SemaphoreType.DMA((_DEPTH,)),
            pltpu.SemaphoreType.DMA((2,)),
        ],
        compiler_params=pltpu.CompilerParams(
            fuse_transposed_lhs_in_matmul=True,
            vmem_limit_bytes=60 * 1024 * 1024,
        ),
    )(xt, lax.slice(xt, (0, _TSTART + _TALIGN), (k, m)), W)


# speedup vs baseline: 1.0427x; 1.0427x over previous
import jax
import jax.numpy as jnp
from jax import lax
from jax.experimental import pallas as pl
from jax.experimental.pallas import tpu as pltpu

_BM = 8192
_NFULL = 12
_TSTART = _NFULL * _BM
_TROWS = 100000 - _TSTART
_TALIGN = 1664
_TREM = _TROWS - _TALIGN
_DEPTH = 6


def _in_copy(xt_hbm, xbuf, insem, chunk, slot):
    return pltpu.make_async_copy(
        xt_hbm.at[:, pl.ds(chunk * _BM, _BM)], xbuf.at[slot], insem.at[slot])


def _out_copy(o_hbm, obuf, outsem, chunk, slot):
    return pltpu.make_async_copy(
        obuf.at[slot], o_hbm.at[pl.ds(chunk * _BM, _BM), :], outsem.at[slot])


def _dot_t(x, w):
    return lax.dot_general(x, w, dimension_numbers=(((0,), (0,)), ((), ())),
                           preferred_element_type=jnp.float32)


def _pipeline(xt_hbm, xrem_vmem, w_vmem, o_hbm, xbuf, obuf, xtail, otail,
              insem, outsem, tsem):
    w = w_vmem[...]

    tail_in = pltpu.make_async_copy(
        xt_hbm.at[:, pl.ds(_TSTART, _TALIGN)], xtail, tsem.at[1])
    tail_in.start()

    for d in range(_DEPTH):
        _in_copy(xt_hbm, xbuf, insem, d, d).start()

    tail_out = pltpu.make_async_copy(
        otail, o_hbm.at[pl.ds(_TSTART, _TROWS), :], tsem.at[0])

    def step(i, carry):
        slot = jax.lax.rem(i, _DEPTH)
        _in_copy(xt_hbm, xbuf, insem, i, slot).wait()

        @pl.when(i >= _DEPTH)
        def _():
            _out_copy(o_hbm, obuf, outsem, i - _DEPTH, slot).wait()

        obuf[slot] = _dot_t(xbuf[slot], w)
        _out_copy(o_hbm, obuf, outsem, i, slot).start()

        @pl.when(i + _DEPTH < _NFULL)
        def _():
            _in_copy(xt_hbm, xbuf, insem, i + _DEPTH, slot).start()

        return carry

    jax.lax.fori_loop(0, _NFULL, step, 0)

    tail_in.wait()
    otail[pl.ds(0, _TALIGN), :] = _dot_t(xtail[...], w)
    otail[pl.ds(_TALIGN, _TREM), :] = _dot_t(xrem_vmem[...], w)
    tail_out.start()

    for d in range(_DEPTH):
        chunk = _NFULL - _DEPTH + d
        _out_copy(o_hbm, obuf, outsem, chunk, chunk % _DEPTH).wait()
    tail_out.wait()


@jax.jit
def kernel(one_hot_atomic, W):
    m, k = one_hot_atomic.shape
    n = W.shape[1]
    xt = one_hot_atomic.T
    return pl.pallas_call(
        _pipeline,
        in_specs=[
            pl.BlockSpec(memory_space=pltpu.MemorySpace.HBM),
            pl.BlockSpec((k, _TREM), lambda: (0, 0)),
            pl.BlockSpec((k, n), lambda: (0, 0)),
        ],
        out_specs=pl.BlockSpec(memory_space=pltpu.MemorySpace.HBM),
        out_shape=jax.ShapeDtypeStruct((m, n), jnp.float32),
        scratch_shapes=[
            pltpu.VMEM((_DEPTH, k, _BM), jnp.float32),
            pltpu.VMEM((_DEPTH, _BM, n), jnp.float32),
            pltpu.VMEM((k, _TALIGN), jnp.float32),
            pltpu.VMEM((_TROWS, n), jnp.float32),
            pltpu.SemaphoreType.DMA((_DEPTH,)),
            pltpu.SemaphoreType.DMA((_DEPTH,)),
            pltpu.SemaphoreType.DMA((2,)),
        ],
        compiler_params=pltpu.CompilerParams(
            fuse_transposed_lhs_in_matmul=True,
            vmem_limit_bytes=60 * 1024 * 1024,
        ),
    )(xt, lax.slice(xt, (0, _TSTART + _TALIGN), (k, m)), W)
